# Initial kernel scaffold; baseline (speedup 1.0000x reference)
import jax
import jax.numpy as jnp
from jax.experimental import pallas as pl


def kernel(x, edge_index, edge_attr, W1, W2, att_l, att_r, bias):
    raise NotImplementedError("write your pallas kernel here")


# speedup vs baseline: 7.3225x; 7.3225x over previous
import functools

import jax
import jax.numpy as jnp
from jax import lax
from jax.experimental import pallas as pl
from jax.experimental.pallas import tpu as pltpu
from jax.experimental.pallas import tpu_sc as plsc

_SLOPE = 0.01
_NUM_SC = 2
_NUM_TILES = 16
_NW = _NUM_SC * _NUM_TILES


def _leaky(v):
    return jnp.where(v >= 0, v, _SLOPE * v)


def _node_body(x_ref, w1a_ref, attr_ref, p_ref, sr_ref):
    xb = x_ref[...]
    p_ref[...] = jnp.dot(xb, w1a_ref[...], preferred_element_type=jnp.float32)
    sr_ref[0, 0, :] = jnp.sum(xb * attr_ref[...], axis=1)


def _edge_body(pj_ref, ea_ref, srd_ref, w1b_ref, attl_ref, mw_ref, ex_ref):
    q = jnp.dot(ea_ref[...], w1b_ref[...], preferred_element_type=jnp.float32)
    m = _leaky(pj_ref[...] + q)
    alpha = _leaky(jnp.sum(m * attl_ref[...], axis=1) + srd_ref[0, 0, :])
    ex = jnp.exp(alpha)
    ex_ref[0, 0, :] = ex
    mw_ref[...] = m * ex[:, None]


def _final_body(acc_ref, den_ref, w2_ref, b_ref, out_ref):
    acc = acc_ref[0] + acc_ref[1]
    den = den_ref[0, 0, 0, :] + den_ref[1, 0, 0, :]
    s = acc / (den + 1e-16)[:, None]
    out_ref[...] = (
        jnp.dot(s, w2_ref[...], preferred_element_type=jnp.float32) + b_ref[...]
    )


def _tc_node(x, w1a, att_r, nb):
    n, d = x.shape
    g = n // nb
    return pl.pallas_call(
        _node_body,
        grid=(g,),
        in_specs=[
            pl.BlockSpec((nb, d), lambda i: (i, 0)),
            pl.BlockSpec((d, d), lambda i: (0, 0)),
            pl.BlockSpec((1, d), lambda i: (0, 0)),
        ],
        out_specs=[
            pl.BlockSpec((nb, d), lambda i: (i, 0)),
            pl.BlockSpec((1, 1, nb), lambda i: (i, 0, 0)),
        ],
        out_shape=[
            jax.ShapeDtypeStruct((n, d), jnp.float32),
            jax.ShapeDtypeStruct((g, 1, nb), jnp.float32),
        ],
    )(x, w1a, att_r)


def _tc_edge(pj, ea, srd3, w1b, attl, eb):
    e, d = pj.shape
    de = ea.shape[1]
    g = e // eb
    return pl.pallas_call(
        _edge_body,
        grid=(g,),
        in_specs=[
            pl.BlockSpec((eb, d), lambda i: (i, 0)),
            pl.BlockSpec((eb, de), lambda i: (i, 0)),
            pl.BlockSpec((1, 1, eb), lambda i: (i, 0, 0)),
            pl.BlockSpec((de, d), lambda i: (0, 0)),
            pl.BlockSpec((1, d), lambda i: (0, 0)),
        ],
        out_specs=[
            pl.BlockSpec((eb, d), lambda i: (i, 0)),
            pl.BlockSpec((1, 1, eb), lambda i: (i, 0, 0)),
        ],
        out_shape=[
            jax.ShapeDtypeStruct((e, d), jnp.float32),
            jax.ShapeDtypeStruct((g, 1, eb), jnp.float32),
        ],
    )(pj, ea, srd3, w1b, attl)


def _tc_final(accp, denp4, w2, bias2, nb):
    _, n, d = accp.shape
    g = n // nb
    return pl.pallas_call(
        _final_body,
        grid=(g,),
        in_specs=[
            pl.BlockSpec((2, nb, d), lambda i: (0, i, 0)),
            pl.BlockSpec((2, 1, 1, nb), lambda i: (0, i, 0, 0)),
            pl.BlockSpec((d, d), lambda i: (0, 0)),
            pl.BlockSpec((1, d), lambda i: (0, 0)),
        ],
        out_specs=pl.BlockSpec((nb, d), lambda i: (i, 0)),
        out_shape=jax.ShapeDtypeStruct((n, d), jnp.float32),
    )(accp, denp4, w2, bias2)


def _sc_mesh():
    return plsc.VectorSubcoreMesh(
        core_axis_name="c", subcore_axis_name="s",
        num_cores=_NUM_SC, num_subcores=_NUM_TILES,
    )


@functools.lru_cache(maxsize=None)
def _make_sc_gather(n, e, d, c):
    epw = e // _NW
    nch = epw // c

    @functools.partial(
        pl.kernel,
        out_type=[
            jax.ShapeDtypeStruct((e, d), jnp.float32),
            jax.ShapeDtypeStruct((e,), jnp.float32),
        ],
        mesh=_sc_mesh(),
        scratch_types=[
            pltpu.VMEM((c,), jnp.int32),
            pltpu.VMEM((c, d), jnp.float32),
            pltpu.VMEM((c,), jnp.int32),
            pltpu.VMEM((c,), jnp.float32),
            pltpu.SemaphoreType.DMA,
        ],
    )
    def sc_gather(p_hbm, sr_hbm, src_hbm, dst_hbm, pj_hbm, srd_hbm,
                  sidx, rows, didx, srv, sem):
        wid = lax.axis_index("s") * _NUM_SC + lax.axis_index("c")
        base0 = wid * epw

        def body(j, carry):
            base = base0 + j * c
            pltpu.sync_copy(src_hbm.at[pl.ds(base, c)], sidx)
            pltpu.async_copy(p_hbm.at[sidx], rows, sem).wait()
            pltpu.sync_copy(rows, pj_hbm.at[pl.ds(base, c)])
            pltpu.sync_copy(dst_hbm.at[pl.ds(base, c)], didx)
            pltpu.async_copy(sr_hbm.at[didx], srv, sem).wait()
            pltpu.sync_copy(srv, srd_hbm.at[pl.ds(base, c)])
            return carry

        lax.fori_loop(0, nch, body, 0)

    return sc_gather


@functools.lru_cache(maxsize=None)
def _make_sc_scatter(npad, e, d, c):
    epw = e // _NW
    nch = epw // c
    npt = npad // _NUM_TILES
    rc = 128
    nrc = npt // rc

    @functools.partial(
        pl.kernel,
        out_type=[
            jax.ShapeDtypeStruct((_NUM_SC, npad, d), jnp.float32),
            jax.ShapeDtypeStruct((_NUM_SC, npad), jnp.float32),
        ],
        mesh=_sc_mesh(),
        scratch_types=[
            pltpu.VMEM((c, d), jnp.float32),
            pltpu.VMEM((c,), jnp.float32),
            pltpu.VMEM((c,), jnp.int32),
            pltpu.VMEM((rc, d), jnp.float32),
            pltpu.VMEM((npad,), jnp.float32),
            pltpu.VMEM_SHARED((npad, d), jnp.float32),
            pltpu.VMEM_SHARED((npad,), jnp.float32),
            pltpu.SemaphoreType.DMA,
        ],
    )
    def sc_scatter(mw_hbm, ex_hbm, dst_hbm, zrows_hbm, zvec_hbm,
                   acc_hbm, den_hbm,
                   mwv, exv, didx, rbuf, dbuf, acc_sh, den_sh, sem):
        cid = lax.axis_index("c")
        sid = lax.axis_index("s")
        wid = sid * _NUM_SC + cid

        def zbody(t, carry):
            off = sid * npt + t * rc
            pltpu.sync_copy(zrows_hbm.at[pl.ds(off, rc)], rbuf)
            pltpu.sync_copy(rbuf, acc_sh.at[pl.ds(off, rc)])
            return carry

        lax.fori_loop(0, nrc, zbody, 0)

        @pl.when(sid == 0)
        def _():
            pltpu.sync_copy(zvec_hbm, dbuf)
            pltpu.sync_copy(dbuf, den_sh)

        plsc.subcore_barrier()

        def body(j, carry):
            base = wid * epw + j * c
            pltpu.sync_copy(dst_hbm.at[pl.ds(base, c)], didx)
            pltpu.sync_copy(mw_hbm.at[pl.ds(base, c)], mwv)
            pltpu.sync_copy(ex_hbm.at[pl.ds(base, c)], exv)
            pltpu.sync_copy(mwv, acc_sh.at[didx], add=True)
            pltpu.sync_copy(exv, den_sh.at[didx], add=True)
            return carry

        lax.fori_loop(0, nch, body, 0)

        plsc.subcore_barrier()

        def ebody(t, carry):
            off = sid * npt + t * rc
            pltpu.sync_copy(acc_sh.at[pl.ds(off, rc)], rbuf)
            pltpu.sync_copy(rbuf, acc_hbm.at[cid, pl.ds(off, rc)])
            return carry

        lax.fori_loop(0, nrc, ebody, 0)

        @pl.when(sid == 0)
        def _():
            pltpu.sync_copy(den_sh, dbuf)
            pltpu.sync_copy(dbuf, den_hbm.at[cid])

    return sc_scatter


def kernel(x, edge_index, edge_attr, W1, W2, att_l, att_r, bias):
    n, d_in = x.shape
    e = edge_index.shape[1]
    d_e = edge_attr.shape[1]
    d_out = W1.shape[1]

    src = edge_index[0]
    dst = edge_index[1]
    w1a = W1[:d_in]
    w1b = W1[d_in:]

    nb = 1000
    p, sr3 = _tc_node(x, w1a, att_r, nb)
    sr = sr3.reshape(n)

    c = 80
    pj, srd = _make_sc_gather(n, e, d_out, c)(p, sr, src, dst)

    eb = 8000
    g = e // eb
    mw, ex3 = _tc_edge(pj, edge_attr, srd.reshape(g, 1, eb), w1b, att_l, eb)
    ex = ex3.reshape(e)

    npad = 10240
    zrows = jnp.zeros((npad, d_out), jnp.float32)
    zvec = jnp.zeros((npad,), jnp.float32)
    accp, denp = _make_sc_scatter(npad, e, d_out, c)(mw, ex, dst, zrows, zvec)

    fb = 1280
    out = _tc_final(
        accp, denp.reshape(_NUM_SC, npad // fb, 1, fb), W2,
        bias.reshape(1, d_out), fb,
    )
    return out[:n]


# speedup vs baseline: 11.6588x; 1.5922x over previous
import functools

import jax
import jax.numpy as jnp
from jax import lax
from jax.experimental import pallas as pl
from jax.experimental.pallas import tpu as pltpu
from jax.experimental.pallas import tpu_sc as plsc

_SLOPE = 0.01
_NUM_SC = 2
_NUM_TILES = 16
_NW = _NUM_SC * _NUM_TILES


def _leaky(v):
    return jnp.where(v >= 0, v, _SLOPE * v)


def _node_body(x_ref, w1a_ref, attr_ref, p_ref, sr_ref):
    xb = x_ref[...]
    p_ref[...] = jnp.dot(xb, w1a_ref[...], preferred_element_type=jnp.float32)
    sr_ref[0, 0, :] = jnp.sum(xb * attr_ref[...], axis=1)


def _edge_body(pj_ref, ea_ref, srd_ref, w1b_ref, attl_ref, mw_ref, ex_ref):
    q = jnp.dot(ea_ref[...], w1b_ref[...], preferred_element_type=jnp.float32)
    m = _leaky(pj_ref[...] + q)
    alpha = _leaky(jnp.sum(m * attl_ref[...], axis=1) + srd_ref[0, 0, :])
    ex = jnp.exp(alpha)
    ex_ref[0, 0, :] = ex
    mw_ref[...] = m * ex[:, None]


def _final_body(acc_ref, den_ref, w2_ref, b_ref, out_ref):
    acc = acc_ref[0] + acc_ref[1]
    den = den_ref[0, 0, 0, :] + den_ref[1, 0, 0, :]
    s = acc / (den + 1e-16)[:, None]
    out_ref[...] = (
        jnp.dot(s, w2_ref[...], preferred_element_type=jnp.float32) + b_ref[...]
    )


def _tc_node(x, w1a, att_r, nb):
    n, d = x.shape
    g = n // nb
    return pl.pallas_call(
        _node_body,
        grid=(g,),
        in_specs=[
            pl.BlockSpec((nb, d), lambda i: (i, 0)),
            pl.BlockSpec((d, d), lambda i: (0, 0)),
            pl.BlockSpec((1, d), lambda i: (0, 0)),
        ],
        out_specs=[
            pl.BlockSpec((nb, d), lambda i: (i, 0)),
            pl.BlockSpec((1, 1, nb), lambda i: (i, 0, 0)),
        ],
        out_shape=[
            jax.ShapeDtypeStruct((n, d), jnp.float32),
            jax.ShapeDtypeStruct((g, 1, nb), jnp.float32),
        ],
    )(x, w1a, att_r)


def _tc_edge(pj, ea, srd3, w1b, attl, eb):
    e, d = pj.shape
    de = ea.shape[1]
    g = e // eb
    return pl.pallas_call(
        _edge_body,
        grid=(g,),
        in_specs=[
            pl.BlockSpec((eb, d), lambda i: (i, 0)),
            pl.BlockSpec((eb, de), lambda i: (i, 0)),
            pl.BlockSpec((1, 1, eb), lambda i: (i, 0, 0)),
            pl.BlockSpec((de, d), lambda i: (0, 0)),
            pl.BlockSpec((1, d), lambda i: (0, 0)),
        ],
        out_specs=[
            pl.BlockSpec((eb, d), lambda i: (i, 0)),
            pl.BlockSpec((1, 1, eb), lambda i: (i, 0, 0)),
        ],
        out_shape=[
            jax.ShapeDtypeStruct((e, d), jnp.float32),
            jax.ShapeDtypeStruct((g, 1, eb), jnp.float32),
        ],
    )(pj, ea, srd3, w1b, attl)


def _tc_final(accp, denp4, w2, bias2, nb):
    _, n, d = accp.shape
    g = n // nb
    return pl.pallas_call(
        _final_body,
        grid=(g,),
        in_specs=[
            pl.BlockSpec((2, nb, d), lambda i: (0, i, 0)),
            pl.BlockSpec((2, 1, 1, nb), lambda i: (0, i, 0, 0)),
            pl.BlockSpec((d, d), lambda i: (0, 0)),
            pl.BlockSpec((1, d), lambda i: (0, 0)),
        ],
        out_specs=pl.BlockSpec((nb, d), lambda i: (i, 0)),
        out_shape=jax.ShapeDtypeStruct((n, d), jnp.float32),
    )(accp, denp4, w2, bias2)


def _sc_mesh():
    return plsc.VectorSubcoreMesh(
        core_axis_name="c", subcore_axis_name="s",
        num_cores=_NUM_SC, num_subcores=_NUM_TILES,
    )


@functools.lru_cache(maxsize=None)
def _make_sc_gather(n, e, d, c):
    epw = e // _NW
    nf = epw // c
    rem = epw - nf * c
    npair = nf // 2
    assert nf % 2 == 0 and rem % 8 == 0

    @functools.partial(
        pl.kernel,
        out_type=[
            jax.ShapeDtypeStruct((e, d), jnp.float32),
            jax.ShapeDtypeStruct((e,), jnp.float32),
        ],
        mesh=_sc_mesh(),
        scratch_types=[
            pltpu.VMEM((epw,), jnp.int32),
            pltpu.VMEM((epw,), jnp.int32),
            pltpu.VMEM((epw,), jnp.float32),
            pltpu.VMEM((c, d), jnp.float32),
            pltpu.VMEM((c, d), jnp.float32),
            pltpu.SemaphoreType.DMA,
            pltpu.SemaphoreType.DMA,
            pltpu.SemaphoreType.DMA,
            pltpu.SemaphoreType.DMA,
            pltpu.SemaphoreType.DMA,
        ],
    )
    def sc_gather(p_hbm, sr_hbm, src_hbm, dst_hbm, pj_hbm, srd_hbm,
                  sidx_all, didx_all, srv_all, rows_a, rows_b,
                  gs_a, gs_b, ws_a, ws_b, ssem):
        wid = lax.axis_index("s") * _NUM_SC + lax.axis_index("c")
        base0 = wid * epw

        pltpu.sync_copy(src_hbm.at[pl.ds(base0, epw)], sidx_all)
        pltpu.sync_copy(dst_hbm.at[pl.ds(base0, epw)], didx_all)

        def sr_fire(j, carry):
            sl = pl.ds(j * c, c)
            pltpu.async_copy(sr_hbm.at[didx_all.at[sl]], srv_all.at[sl], ssem)
            return carry

        lax.fori_loop(0, nf, sr_fire, 0)
        if rem:
            sl = pl.ds(nf * c, rem)
            pltpu.async_copy(sr_hbm.at[didx_all.at[sl]], srv_all.at[sl], ssem)

        def pair(jj, carry):
            j0 = jj * 2
            j1 = j0 + 1

            @pl.when(jj > 0)
            def _():
                pltpu.make_async_copy(
                    rows_a, pj_hbm.at[pl.ds(base0 + (j0 - 2) * c, c)], ws_a
                ).wait()
                pltpu.make_async_copy(
                    rows_b, pj_hbm.at[pl.ds(base0 + (j1 - 2) * c, c)], ws_b
                ).wait()

            sl0 = sidx_all.at[pl.ds(j0 * c, c)]
            sl1 = sidx_all.at[pl.ds(j1 * c, c)]
            pltpu.async_copy(p_hbm.at[sl0], rows_a, gs_a)
            pltpu.async_copy(p_hbm.at[sl1], rows_b, gs_b)
            pltpu.make_async_copy(p_hbm.at[sl0], rows_a, gs_a).wait()
            pltpu.async_copy(rows_a, pj_hbm.at[pl.ds(base0 + j0 * c, c)], ws_a)
            pltpu.make_async_copy(p_hbm.at[sl1], rows_b, gs_b).wait()
            pltpu.async_copy(rows_b, pj_hbm.at[pl.ds(base0 + j1 * c, c)], ws_b)
            return carry

        lax.fori_loop(0, npair, pair, 0)
        pltpu.make_async_copy(
            rows_a, pj_hbm.at[pl.ds(base0 + (nf - 2) * c, c)], ws_a
        ).wait()
        pltpu.make_async_copy(
            rows_b, pj_hbm.at[pl.ds(base0 + (nf - 1) * c, c)], ws_b
        ).wait()

        if rem:
            slr = sidx_all.at[pl.ds(nf * c, rem)]
            rr = rows_a.at[pl.ds(0, rem)]
            pltpu.async_copy(p_hbm.at[slr], rr, gs_a).wait()
            pltpu.sync_copy(rr, pj_hbm.at[pl.ds(base0 + nf * c, rem)])

        pltpu.make_async_copy(sr_hbm.at[didx_all], srv_all, ssem).wait()
        pltpu.sync_copy(srv_all, srd_hbm.at[pl.ds(base0, epw)])

    return sc_gather


@functools.lru_cache(maxsize=None)
def _make_sc_scatter(npad, e, d, c):
    epw = e // _NW
    npt = npad // _NUM_TILES
    rc = 64
    nrc = npt // rc
    dc = 1280
    ndc = npad // dc

    nf = epw // c
    rem = epw - nf * c
    npair = nf // 2
    assert nf % 2 == 0 and rem % 8 == 0

    @functools.partial(
        pl.kernel,
        out_type=[
            jax.ShapeDtypeStruct((_NUM_SC, npad, d), jnp.float32),
            jax.ShapeDtypeStruct((_NUM_SC, npad), jnp.float32),
        ],
        mesh=_sc_mesh(),
        scratch_types=[
            pltpu.VMEM((c, d), jnp.float32),
            pltpu.VMEM((c, d), jnp.float32),
            pltpu.VMEM((c,), jnp.int32),
            pltpu.VMEM((c,), jnp.int32),
            pltpu.VMEM((max(rem, 8),), jnp.int32),
            pltpu.VMEM((c,), jnp.float32),
            pltpu.VMEM((c,), jnp.float32),
            pltpu.VMEM((rc, d), jnp.float32),
            pltpu.VMEM((dc,), jnp.float32),
            pltpu.VMEM_SHARED((npad, d), jnp.float32),
            pltpu.VMEM_SHARED((npad,), jnp.float32),
            pltpu.SemaphoreType.DMA,
            pltpu.SemaphoreType.DMA,
            pltpu.SemaphoreType.DMA,
            pltpu.SemaphoreType.DMA,
            pltpu.SemaphoreType.DMA,
            pltpu.SemaphoreType.DMA,
        ],
    )
    def sc_scatter(mw_hbm, ex_hbm, dst_hbm, acc_hbm, den_hbm,
                   mw_a, mw_b, didx_a, didx_b, didx_r, ex_a, ex_b, rbuf, dbuf,
                   acc_sh, den_sh,
                   ls_a, ls_b, is_a, is_b, ss_a, ss_b):
        cid = lax.axis_index("c")
        sid = lax.axis_index("s")
        wid = sid * _NUM_SC + cid
        base0 = wid * epw
        zv = jnp.zeros((16,), jnp.float32)

        def zrow(i, carry):
            for k in range(d // 16):
                rbuf[i, pl.ds(k * 16, 16)] = zv
            return carry

        lax.fori_loop(0, rc, zrow, 0)

        def zbody(t, carry):
            pltpu.sync_copy(rbuf, acc_sh.at[pl.ds(sid * npt + t * rc, rc)])
            return carry

        lax.fori_loop(0, nrc, zbody, 0)

        @pl.when(sid == 0)
        def _():
            def zd(i, carry):
                dbuf[pl.ds(i * 16, 16)] = zv
                return carry

            lax.fori_loop(0, dc // 16, zd, 0)

            def zden(k, carry):
                pltpu.sync_copy(dbuf, den_sh.at[pl.ds(k * dc, dc)])
                return carry

            lax.fori_loop(0, ndc, zden, 0)

        plsc.subcore_barrier()

        def pair(jj, carry):
            j0 = jj * 2
            j1 = j0 + 1

            @pl.when(jj > 0)
            def _():
                pltpu.make_async_copy(mw_a, acc_sh.at[didx_a], ss_a).wait()
                pltpu.make_async_copy(ex_a, den_sh.at[didx_a], ss_a).wait()
                pltpu.make_async_copy(mw_b, acc_sh.at[didx_b], ss_b).wait()
                pltpu.make_async_copy(ex_b, den_sh.at[didx_b], ss_b).wait()

            sl0 = pl.ds(base0 + j0 * c, c)
            sl1 = pl.ds(base0 + j1 * c, c)
            pltpu.async_copy(dst_hbm.at[sl0], didx_a, is_a)
            pltpu.async_copy(mw_hbm.at[sl0], mw_a, ls_a)
            pltpu.async_copy(ex_hbm.at[sl0], ex_a, ls_a)
            pltpu.async_copy(dst_hbm.at[sl1], didx_b, is_b)
            pltpu.async_copy(mw_hbm.at[sl1], mw_b, ls_b)
            pltpu.async_copy(ex_hbm.at[sl1], ex_b, ls_b)

            pltpu.make_async_copy(dst_hbm.at[sl0], didx_a, is_a).wait()
            pltpu.make_async_copy(mw_hbm.at[sl0], mw_a, ls_a).wait()
            pltpu.make_async_copy(ex_hbm.at[sl0], ex_a, ls_a).wait()
            pltpu.async_copy(mw_a, acc_sh.at[didx_a], ss_a, add=True)
            pltpu.async_copy(ex_a, den_sh.at[didx_a], ss_a, add=True)

            pltpu.make_async_copy(dst_hbm.at[sl1], didx_b, is_b).wait()
            pltpu.make_async_copy(mw_hbm.at[sl1], mw_b, ls_b).wait()
            pltpu.make_async_copy(ex_hbm.at[sl1], ex_b, ls_b).wait()
            pltpu.async_copy(mw_b, acc_sh.at[didx_b], ss_b, add=True)
            pltpu.async_copy(ex_b, den_sh.at[didx_b], ss_b, add=True)
            return carry

        lax.fori_loop(0, npair, pair, 0)
        pltpu.make_async_copy(mw_a, acc_sh.at[didx_a], ss_a).wait()
        pltpu.make_async_copy(ex_a, den_sh.at[didx_a], ss_a).wait()
        pltpu.make_async_copy(mw_b, acc_sh.at[didx_b], ss_b).wait()
        pltpu.make_async_copy(ex_b, den_sh.at[didx_b], ss_b).wait()

        if rem:
            slr = pl.ds(base0 + nf * c, rem)
            mr = mw_a.at[pl.ds(0, rem)]
            xr = ex_a.at[pl.ds(0, rem)]
            pltpu.sync_copy(dst_hbm.at[slr], didx_r)
            pltpu.sync_copy(mw_hbm.at[slr], mr)
            pltpu.sync_copy(ex_hbm.at[slr], xr)
            pltpu.sync_copy(mr, acc_sh.at[didx_r], add=True)
            pltpu.sync_copy(xr, den_sh.at[didx_r], add=True)

        plsc.subcore_barrier()

        def ebody(t, carry):
            off = sid * npt + t * rc
            pltpu.sync_copy(acc_sh.at[pl.ds(off, rc)], rbuf)
            pltpu.sync_copy(rbuf, acc_hbm.at[cid, pl.ds(off, rc)])
            return carry

        lax.fori_loop(0, nrc, ebody, 0)

        @pl.when(sid == 0)
        def _():
            def eden(k, carry):
                sl = pl.ds(k * dc, dc)
                pltpu.sync_copy(den_sh.at[sl], dbuf)
                pltpu.sync_copy(dbuf, den_hbm.at[cid, sl])
                return carry

            lax.fori_loop(0, ndc, eden, 0)

    return sc_scatter


def kernel(x, edge_index, edge_attr, W1, W2, att_l, att_r, bias):
    n, d_in = x.shape
    e = edge_index.shape[1]
    d_e = edge_attr.shape[1]
    d_out = W1.shape[1]

    src = edge_index[0]
    dst = edge_index[1]
    w1a = W1[:d_in]
    w1b = W1[d_in:]

    nb = 1000
    p, sr3 = _tc_node(x, w1a, att_r, nb)
    sr = sr3.reshape(n)

    c = 128
    pj, srd = _make_sc_gather(n, e, d_out, c)(p, sr, src, dst)

    eb = 8000
    g = e // eb
    mw, ex3 = _tc_edge(pj, edge_attr, srd.reshape(g, 1, eb), w1b, att_l, eb)
    ex = ex3.reshape(e)

    npad = 10240
    accp, denp = _make_sc_scatter(npad, e, d_out, c)(mw, ex, dst)

    fb = 1280
    out = _tc_final(
        accp, denp.reshape(_NUM_SC, npad // fb, 1, fb), W2,
        bias.reshape(1, d_out), fb,
    )
    return out[:n]


# speedup vs baseline: 13.7934x; 1.1831x over previous
import functools

import jax
import jax.numpy as jnp
from jax import lax
from jax.experimental import pallas as pl
from jax.experimental.pallas import tpu as pltpu
from jax.experimental.pallas import tpu_sc as plsc

_SLOPE = 0.01
_NUM_SC = 2
_NUM_TILES = 16
_NW = _NUM_SC * _NUM_TILES


def _leaky(v):
    return jnp.maximum(v, _SLOPE * v)


def _node_body(x_ref, w1a_ref, attr_ref, p_ref, sr_ref):
    xb = x_ref[...]
    p_ref[...] = jnp.dot(xb, w1a_ref[...], preferred_element_type=jnp.float32)
    sr_ref[0, 0, :] = jnp.sum(xb * attr_ref[...], axis=1)


def _edge_body(pj_ref, ea_ref, srd_ref, w1b_ref, attl_ref, mw_ref, ex_ref):
    q = jnp.dot(ea_ref[...], w1b_ref[...], preferred_element_type=jnp.float32)
    m = _leaky(pj_ref[...] + q)
    aj = jax.lax.dot_general(
        attl_ref[...], m, (((1,), (1,)), ((), ())),
        preferred_element_type=jnp.float32,
    )
    alpha = _leaky(aj[0, :] + srd_ref[0, 0, :])
    ex = jnp.exp(alpha)
    ex_ref[0, 0, :] = ex
    mw_ref[...] = m * ex[:, None]


def _final_body(acc_ref, den_ref, w2_ref, b_ref, out_ref):
    acc = acc_ref[0] + acc_ref[1]
    den = den_ref[0, 0, 0, :] + den_ref[1, 0, 0, :]
    s = acc / (den + 1e-16)[:, None]
    out_ref[...] = (
        jnp.dot(s, w2_ref[...], preferred_element_type=jnp.float32) + b_ref[...]
    )


def _tc_node(x, w1a, att_r, nb):
    n, d = x.shape
    g = n // nb
    return pl.pallas_call(
        _node_body,
        grid=(g,),
        in_specs=[
            pl.BlockSpec((nb, d), lambda i: (i, 0)),
            pl.BlockSpec((d, d), lambda i: (0, 0)),
            pl.BlockSpec((1, d), lambda i: (0, 0)),
        ],
        out_specs=[
            pl.BlockSpec((nb, d), lambda i: (i, 0)),
            pl.BlockSpec((1, 1, nb), lambda i: (i, 0, 0)),
        ],
        out_shape=[
            jax.ShapeDtypeStruct((n, d), jnp.float32),
            jax.ShapeDtypeStruct((g, 1, nb), jnp.float32),
        ],
    )(x, w1a, att_r)


def _tc_edge(pj, ea, srd3, w1b, attl, eb):
    e, d = pj.shape
    de = ea.shape[1]
    g = e // eb
    return pl.pallas_call(
        _edge_body,
        grid=(g,),
        in_specs=[
            pl.BlockSpec((eb, d), lambda i: (i, 0)),
            pl.BlockSpec((eb, de), lambda i: (i, 0)),
            pl.BlockSpec((1, 1, eb), lambda i: (i, 0, 0)),
            pl.BlockSpec((de, d), lambda i: (0, 0)),
            pl.BlockSpec((1, d), lambda i: (0, 0)),
        ],
        out_specs=[
            pl.BlockSpec((eb, d), lambda i: (i, 0)),
            pl.BlockSpec((1, 1, eb), lambda i: (i, 0, 0)),
        ],
        out_shape=[
            jax.ShapeDtypeStruct((e, d), jnp.float32),
            jax.ShapeDtypeStruct((g, 1, eb), jnp.float32),
        ],
    )(pj, ea, srd3, w1b, attl)


def _tc_final(accp, denp4, w2, bias2, nb):
    _, n, d = accp.shape
    g = n // nb
    return pl.pallas_call(
        _final_body,
        grid=(g,),
        in_specs=[
            pl.BlockSpec((2, nb, d), lambda i: (0, i, 0)),
            pl.BlockSpec((2, 1, 1, nb), lambda i: (0, i, 0, 0)),
            pl.BlockSpec((d, d), lambda i: (0, 0)),
            pl.BlockSpec((1, d), lambda i: (0, 0)),
        ],
        out_specs=pl.BlockSpec((nb, d), lambda i: (i, 0)),
        out_shape=jax.ShapeDtypeStruct((n, d), jnp.float32),
    )(accp, denp4, w2, bias2)


def _sc_mesh():
    return plsc.VectorSubcoreMesh(
        core_axis_name="c", subcore_axis_name="s",
        num_cores=_NUM_SC, num_subcores=_NUM_TILES,
    )


@functools.lru_cache(maxsize=None)
def _make_sc_gather(n, e, d, c):
    epw = e // _NW
    nf = epw // c
    rem = epw - nf * c
    npair = nf // 2
    assert nf % 2 == 0 and rem % 8 == 0

    @functools.partial(
        pl.kernel,
        out_type=[
            jax.ShapeDtypeStruct((e, d), jnp.float32),
            jax.ShapeDtypeStruct((e,), jnp.float32),
        ],
        mesh=_sc_mesh(),
        scratch_types=[
            pltpu.VMEM((epw,), jnp.int32),
            pltpu.VMEM((epw,), jnp.int32),
            pltpu.VMEM((epw,), jnp.float32),
            pltpu.VMEM((c, d), jnp.float32),
            pltpu.VMEM((c, d), jnp.float32),
            pltpu.SemaphoreType.DMA,
            pltpu.SemaphoreType.DMA,
            pltpu.SemaphoreType.DMA,
            pltpu.SemaphoreType.DMA,
            pltpu.SemaphoreType.DMA,
        ],
    )
    def sc_gather(p_hbm, sr_hbm, src_hbm, dst_hbm, pj_hbm, srd_hbm,
                  sidx_all, didx_all, srv_all, rows_a, rows_b,
                  gs_a, gs_b, ws_a, ws_b, ssem):
        wid = lax.axis_index("s") * _NUM_SC + lax.axis_index("c")
        base0 = wid * epw

        pltpu.sync_copy(src_hbm.at[pl.ds(base0, epw)], sidx_all)
        pltpu.sync_copy(dst_hbm.at[pl.ds(base0, epw)], didx_all)

        def sr_fire(j, carry):
            sl = pl.ds(j * c, c)
            pltpu.async_copy(sr_hbm.at[didx_all.at[sl]], srv_all.at[sl], ssem)
            return carry

        lax.fori_loop(0, nf, sr_fire, 0)
        if rem:
            sl = pl.ds(nf * c, rem)
            pltpu.async_copy(sr_hbm.at[didx_all.at[sl]], srv_all.at[sl], ssem)

        def pair(jj, carry):
            j0 = jj * 2
            j1 = j0 + 1

            @pl.when(jj > 0)
            def _():
                pltpu.make_async_copy(
                    rows_a, pj_hbm.at[pl.ds(base0 + (j0 - 2) * c, c)], ws_a
                ).wait()
                pltpu.make_async_copy(
                    rows_b, pj_hbm.at[pl.ds(base0 + (j1 - 2) * c, c)], ws_b
                ).wait()

            sl0 = sidx_all.at[pl.ds(j0 * c, c)]
            sl1 = sidx_all.at[pl.ds(j1 * c, c)]
            pltpu.async_copy(p_hbm.at[sl0], rows_a, gs_a)
            pltpu.async_copy(p_hbm.at[sl1], rows_b, gs_b)
            pltpu.make_async_copy(p_hbm.at[sl0], rows_a, gs_a).wait()
            pltpu.async_copy(rows_a, pj_hbm.at[pl.ds(base0 + j0 * c, c)], ws_a)
            pltpu.make_async_copy(p_hbm.at[sl1], rows_b, gs_b).wait()
            pltpu.async_copy(rows_b, pj_hbm.at[pl.ds(base0 + j1 * c, c)], ws_b)
            return carry

        lax.fori_loop(0, npair, pair, 0)
        pltpu.make_async_copy(
            rows_a, pj_hbm.at[pl.ds(base0 + (nf - 2) * c, c)], ws_a
        ).wait()
        pltpu.make_async_copy(
            rows_b, pj_hbm.at[pl.ds(base0 + (nf - 1) * c, c)], ws_b
        ).wait()

        if rem:
            slr = sidx_all.at[pl.ds(nf * c, rem)]
            rr = rows_a.at[pl.ds(0, rem)]
            pltpu.async_copy(p_hbm.at[slr], rr, gs_a).wait()
            pltpu.sync_copy(rr, pj_hbm.at[pl.ds(base0 + nf * c, rem)])

        pltpu.make_async_copy(sr_hbm.at[didx_all], srv_all, ssem).wait()
        pltpu.sync_copy(srv_all, srd_hbm.at[pl.ds(base0, epw)])

    return sc_gather


@functools.lru_cache(maxsize=None)
def _make_sc_scatter(npad, e, d, c):
    epw = e // _NW
    npt = npad // _NUM_TILES
    rc = 64
    nrc = npt // rc
    dc = 1280
    ndc = npad // dc

    nf = epw // c
    rem = epw - nf * c
    npair = nf // 2
    assert nf % 2 == 0 and rem % 8 == 0

    @functools.partial(
        pl.kernel,
        out_type=[
            jax.ShapeDtypeStruct((_NUM_SC, npad, d), jnp.float32),
            jax.ShapeDtypeStruct((_NUM_SC, npad), jnp.float32),
        ],
        mesh=_sc_mesh(),
        scratch_types=[
            pltpu.VMEM((c, d), jnp.float32),
            pltpu.VMEM((c, d), jnp.float32),
            pltpu.VMEM((c,), jnp.int32),
            pltpu.VMEM((c,), jnp.int32),
            pltpu.VMEM((max(rem, 8),), jnp.int32),
            pltpu.VMEM((c,), jnp.float32),
            pltpu.VMEM((c,), jnp.float32),
            pltpu.VMEM((rc, d), jnp.float32),
            pltpu.VMEM((dc,), jnp.float32),
            pltpu.VMEM_SHARED((npad, d), jnp.float32),
            pltpu.VMEM_SHARED((npad,), jnp.float32),
            pltpu.SemaphoreType.DMA,
            pltpu.SemaphoreType.DMA,
            pltpu.SemaphoreType.DMA,
            pltpu.SemaphoreType.DMA,
            pltpu.SemaphoreType.DMA,
            pltpu.SemaphoreType.DMA,
        ],
    )
    def sc_scatter(mw_hbm, ex_hbm, dst_hbm, acc_hbm, den_hbm,
                   mw_a, mw_b, didx_a, didx_b, didx_r, ex_a, ex_b, rbuf, dbuf,
                   acc_sh, den_sh,
                   ls_a, ls_b, is_a, is_b, ss_a, ss_b):
        cid = lax.axis_index("c")
        sid = lax.axis_index("s")
        wid = sid * _NUM_SC + cid
        base0 = wid * epw
        zv = jnp.zeros((16,), jnp.float32)

        def zrow(i, carry):
            for k in range(d // 16):
                rbuf[i, pl.ds(k * 16, 16)] = zv
            return carry

        lax.fori_loop(0, rc, zrow, 0)

        def zbody(t, carry):
            pltpu.sync_copy(rbuf, acc_sh.at[pl.ds(sid * npt + t * rc, rc)])
            return carry

        lax.fori_loop(0, nrc, zbody, 0)

        @pl.when(sid == 0)
        def _():
            def zd(i, carry):
                dbuf[pl.ds(i * 16, 16)] = zv
                return carry

            lax.fori_loop(0, dc // 16, zd, 0)

            def zden(k, carry):
                pltpu.sync_copy(dbuf, den_sh.at[pl.ds(k * dc, dc)])
                return carry

            lax.fori_loop(0, ndc, zden, 0)

        plsc.subcore_barrier()

        def pair(jj, carry):
            j0 = jj * 2
            j1 = j0 + 1

            @pl.when(jj > 0)
            def _():
                pltpu.make_async_copy(mw_a, acc_sh.at[didx_a], ss_a).wait()
                pltpu.make_async_copy(ex_a, den_sh.at[didx_a], ss_a).wait()
                pltpu.make_async_copy(mw_b, acc_sh.at[didx_b], ss_b).wait()
                pltpu.make_async_copy(ex_b, den_sh.at[didx_b], ss_b).wait()

            sl0 = pl.ds(base0 + j0 * c, c)
            sl1 = pl.ds(base0 + j1 * c, c)
            pltpu.async_copy(dst_hbm.at[sl0], didx_a, is_a)
            pltpu.async_copy(mw_hbm.at[sl0], mw_a, ls_a)
            pltpu.async_copy(ex_hbm.at[sl0], ex_a, ls_a)
            pltpu.async_copy(dst_hbm.at[sl1], didx_b, is_b)
            pltpu.async_copy(mw_hbm.at[sl1], mw_b, ls_b)
            pltpu.async_copy(ex_hbm.at[sl1], ex_b, ls_b)

            pltpu.make_async_copy(dst_hbm.at[sl0], didx_a, is_a).wait()
            pltpu.make_async_copy(mw_hbm.at[sl0], mw_a, ls_a).wait()
            pltpu.make_async_copy(ex_hbm.at[sl0], ex_a, ls_a).wait()
            pltpu.async_copy(mw_a, acc_sh.at[didx_a], ss_a, add=True)
            pltpu.async_copy(ex_a, den_sh.at[didx_a], ss_a, add=True)

            pltpu.make_async_copy(dst_hbm.at[sl1], didx_b, is_b).wait()
            pltpu.make_async_copy(mw_hbm.at[sl1], mw_b, ls_b).wait()
            pltpu.make_async_copy(ex_hbm.at[sl1], ex_b, ls_b).wait()
            pltpu.async_copy(mw_b, acc_sh.at[didx_b], ss_b, add=True)
            pltpu.async_copy(ex_b, den_sh.at[didx_b], ss_b, add=True)
            return carry

        lax.fori_loop(0, npair, pair, 0)
        pltpu.make_async_copy(mw_a, acc_sh.at[didx_a], ss_a).wait()
        pltpu.make_async_copy(ex_a, den_sh.at[didx_a], ss_a).wait()
        pltpu.make_async_copy(mw_b, acc_sh.at[didx_b], ss_b).wait()
        pltpu.make_async_copy(ex_b, den_sh.at[didx_b], ss_b).wait()

        if rem:
            slr = pl.ds(base0 + nf * c, rem)
            mr = mw_a.at[pl.ds(0, rem)]
            xr = ex_a.at[pl.ds(0, rem)]
            pltpu.sync_copy(dst_hbm.at[slr], didx_r)
            pltpu.sync_copy(mw_hbm.at[slr], mr)
            pltpu.sync_copy(ex_hbm.at[slr], xr)
            pltpu.sync_copy(mr, acc_sh.at[didx_r], add=True)
            pltpu.sync_copy(xr, den_sh.at[didx_r], add=True)

        plsc.subcore_barrier()

        def ebody(t, carry):
            off = sid * npt + t * rc
            pltpu.sync_copy(acc_sh.at[pl.ds(off, rc)], rbuf)
            pltpu.sync_copy(rbuf, acc_hbm.at[cid, pl.ds(off, rc)])
            return carry

        lax.fori_loop(0, nrc, ebody, 0)

        @pl.when(sid == 0)
        def _():
            def eden(k, carry):
                sl = pl.ds(k * dc, dc)
                pltpu.sync_copy(den_sh.at[sl], dbuf)
                pltpu.sync_copy(dbuf, den_hbm.at[cid, sl])
                return carry

            lax.fori_loop(0, ndc, eden, 0)

    return sc_scatter


def kernel(x, edge_index, edge_attr, W1, W2, att_l, att_r, bias):
    n, d_in = x.shape
    e = edge_index.shape[1]
    d_e = edge_attr.shape[1]
    d_out = W1.shape[1]

    src = edge_index[0]
    dst = edge_index[1]
    w1a = W1[:d_in]
    w1b = W1[d_in:]

    nb = 1000
    p, sr3 = _tc_node(x, w1a, att_r, nb)
    sr = sr3.reshape(n)

    c = 128
    pj, srd = _make_sc_gather(n, e, d_out, c)(p, sr, src, dst)

    eb = 8000
    g = e // eb
    mw, ex3 = _tc_edge(pj, edge_attr, srd.reshape(g, 1, eb), w1b, att_l, eb)
    ex = ex3.reshape(e)

    npad = 10240
    accp, denp = _make_sc_scatter(npad, e, d_out, c)(mw, ex, dst)

    fb = 1280
    out = _tc_final(
        accp, denp.reshape(_NUM_SC, npad // fb, 1, fb), W2,
        bias.reshape(1, d_out), fb,
    )
    return out[:n]


# speedup vs baseline: 14.7734x; 1.0710x over previous
import functools

import jax
import jax.numpy as jnp
from jax import lax
from jax.experimental import pallas as pl
from jax.experimental.pallas import tpu as pltpu
from jax.experimental.pallas import tpu_sc as plsc

_SLOPE = 0.01
_NUM_SC = 2
_NUM_TILES = 16
_NW = _NUM_SC * _NUM_TILES


def _leaky(v):
    return jnp.maximum(v, _SLOPE * v)


def _node_body(x_ref, w1a_ref, attr_ref, p_ref, sr_ref):
    xb = x_ref[...]
    p_ref[...] = jnp.dot(xb, w1a_ref[...], preferred_element_type=jnp.float32)
    sr_ref[0, 0, :] = jnp.sum(xb * attr_ref[...], axis=1)


def _edge_body(pj_ref, ea_ref, srd_ref, w1b_ref, attl_ref, mw_ref, ex_ref):
    q = jnp.dot(ea_ref[...], w1b_ref[...], preferred_element_type=jnp.float32)
    m = _leaky(pj_ref[...] + q)
    aj = jax.lax.dot_general(
        attl_ref[...], m, (((1,), (1,)), ((), ())),
        preferred_element_type=jnp.float32,
    )
    alpha = _leaky(aj[0, :] + srd_ref[0, 0, :])
    ex = jnp.exp(alpha)
    ex_ref[0, 0, :] = ex
    mw_ref[...] = m * ex[:, None]


def _final_body(acc_ref, den_ref, w2_ref, b_ref, out_ref):
    acc = acc_ref[0] + acc_ref[1]
    den = den_ref[0, 0, 0, :] + den_ref[1, 0, 0, :]
    s = acc / (den + 1e-16)[:, None]
    out_ref[...] = (
        jnp.dot(s, w2_ref[...], preferred_element_type=jnp.float32) + b_ref[...]
    )


def _tc_node(x, w1a, att_r, nb):
    n, d = x.shape
    g = n // nb
    return pl.pallas_call(
        _node_body,
        grid=(g,),
        in_specs=[
            pl.BlockSpec((nb, d), lambda i: (i, 0)),
            pl.BlockSpec((d, d), lambda i: (0, 0)),
            pl.BlockSpec((1, d), lambda i: (0, 0)),
        ],
        out_specs=[
            pl.BlockSpec((nb, d), lambda i: (i, 0)),
            pl.BlockSpec((1, 1, nb), lambda i: (i, 0, 0)),
        ],
        out_shape=[
            jax.ShapeDtypeStruct((n, d), jnp.float32),
            jax.ShapeDtypeStruct((g, 1, nb), jnp.float32),
        ],
    )(x, w1a, att_r)


def _tc_edge(pj, ea, srd3, w1b, attl, eb):
    e, d = pj.shape
    de = ea.shape[1]
    g = e // eb
    return pl.pallas_call(
        _edge_body,
        grid=(g,),
        in_specs=[
            pl.BlockSpec((eb, d), lambda i: (i, 0)),
            pl.BlockSpec((eb, de), lambda i: (i, 0)),
            pl.BlockSpec((1, 1, eb), lambda i: (i, 0, 0)),
            pl.BlockSpec((de, d), lambda i: (0, 0)),
            pl.BlockSpec((1, d), lambda i: (0, 0)),
        ],
        out_specs=[
            pl.BlockSpec((eb, d), lambda i: (i, 0)),
            pl.BlockSpec((1, 1, eb), lambda i: (i, 0, 0)),
        ],
        out_shape=[
            jax.ShapeDtypeStruct((e, d), jnp.float32),
            jax.ShapeDtypeStruct((g, 1, eb), jnp.float32),
        ],
    )(pj, ea, srd3, w1b, attl)


def _tc_final(accp, denp4, w2, bias2, nb):
    _, n, d = accp.shape
    g = n // nb
    return pl.pallas_call(
        _final_body,
        grid=(g,),
        in_specs=[
            pl.BlockSpec((2, nb, d), lambda i: (0, i, 0)),
            pl.BlockSpec((2, 1, 1, nb), lambda i: (0, i, 0, 0)),
            pl.BlockSpec((d, d), lambda i: (0, 0)),
            pl.BlockSpec((1, d), lambda i: (0, 0)),
        ],
        out_specs=pl.BlockSpec((nb, d), lambda i: (i, 0)),
        out_shape=jax.ShapeDtypeStruct((n, d), jnp.float32),
    )(accp, denp4, w2, bias2)


def _sc_mesh():
    return plsc.VectorSubcoreMesh(
        core_axis_name="c", subcore_axis_name="s",
        num_cores=_NUM_SC, num_subcores=_NUM_TILES,
    )


_NBUF = 4


@functools.lru_cache(maxsize=None)
def _make_sc_gather(n, e, d, c):
    epw = e // _NW
    nf = epw // c
    rem = epw - nf * c
    ngrp = nf // _NBUF
    nleft = nf - ngrp * _NBUF
    assert rem % 8 == 0

    @functools.partial(
        pl.kernel,
        out_type=[
            jax.ShapeDtypeStruct((e, d), jnp.float32),
            jax.ShapeDtypeStruct((e,), jnp.float32),
        ],
        mesh=_sc_mesh(),
        scratch_types=[
            pltpu.VMEM((epw,), jnp.int32),
            pltpu.VMEM((epw,), jnp.int32),
            pltpu.VMEM((epw,), jnp.float32),
        ] + [pltpu.VMEM((c, d), jnp.float32)] * _NBUF
          + [pltpu.SemaphoreType.DMA] * (2 * _NBUF + 1),
    )
    def sc_gather(p_hbm, sr_hbm, src_hbm, dst_hbm, pj_hbm, srd_hbm,
                  sidx_all, didx_all, srv_all, *bufsem):
        rows = bufsem[:_NBUF]
        gsem = bufsem[_NBUF:2 * _NBUF]
        wsem = bufsem[2 * _NBUF:3 * _NBUF]
        ssem = bufsem[3 * _NBUF]
        wid = lax.axis_index("s") * _NUM_SC + lax.axis_index("c")
        base0 = wid * epw

        pltpu.sync_copy(src_hbm.at[pl.ds(base0, epw)], sidx_all)
        pltpu.sync_copy(dst_hbm.at[pl.ds(base0, epw)], didx_all)

        def sr_fire(j, carry):
            sl = pl.ds(j * c, c)
            pltpu.async_copy(sr_hbm.at[didx_all.at[sl]], srv_all.at[sl], ssem)
            return carry

        lax.fori_loop(0, nf, sr_fire, 0)
        if rem:
            sl = pl.ds(nf * c, rem)
            pltpu.async_copy(sr_hbm.at[didx_all.at[sl]], srv_all.at[sl], ssem)

        def group(g, carry):
            for b in range(_NBUF):
                j = g * _NBUF + b

                @pl.when(g > 0)
                def _(b=b, j=j):
                    pltpu.make_async_copy(
                        rows[b],
                        pj_hbm.at[pl.ds(base0 + (j - _NBUF) * c, c)],
                        wsem[b],
                    ).wait()

                pltpu.async_copy(
                    p_hbm.at[sidx_all.at[pl.ds(j * c, c)]], rows[b], gsem[b]
                )
            for b in range(_NBUF):
                j = g * _NBUF + b
                pltpu.make_async_copy(
                    p_hbm.at[sidx_all.at[pl.ds(j * c, c)]], rows[b], gsem[b]
                ).wait()
                pltpu.async_copy(
                    rows[b], pj_hbm.at[pl.ds(base0 + j * c, c)], wsem[b]
                )
            return carry

        lax.fori_loop(0, ngrp, group, 0)
        for b in range(_NBUF):
            j = (ngrp - 1) * _NBUF + b
            pltpu.make_async_copy(
                rows[b], pj_hbm.at[pl.ds(base0 + j * c, c)], wsem[b]
            ).wait()

        for b in range(nleft):
            j = ngrp * _NBUF + b
            pltpu.async_copy(
                p_hbm.at[sidx_all.at[pl.ds(j * c, c)]], rows[b], gsem[b]
            )
        for b in range(nleft):
            j = ngrp * _NBUF + b
            pltpu.make_async_copy(
                p_hbm.at[sidx_all.at[pl.ds(j * c, c)]], rows[b], gsem[b]
            ).wait()
            pltpu.sync_copy(rows[b], pj_hbm.at[pl.ds(base0 + j * c, c)])

        if rem:
            slr = sidx_all.at[pl.ds(nf * c, rem)]
            rr = rows[0].at[pl.ds(0, rem)]
            pltpu.async_copy(p_hbm.at[slr], rr, gsem[0]).wait()
            pltpu.sync_copy(rr, pj_hbm.at[pl.ds(base0 + nf * c, rem)])

        pltpu.make_async_copy(sr_hbm.at[didx_all], srv_all, ssem).wait()
        pltpu.sync_copy(srv_all, srd_hbm.at[pl.ds(base0, epw)])

    return sc_gather


@functools.lru_cache(maxsize=None)
def _make_sc_scatter(npad, e, d, c):
    epw = e // _NW
    npt = npad // _NUM_TILES
    rc = 64
    nrc = npt // rc
    dc = 1280
    ndc = npad // dc

    nf = epw // c
    rem = epw - nf * c
    ngrp = nf // _NBUF
    nleft = nf - ngrp * _NBUF
    assert rem % 8 == 0

    @functools.partial(
        pl.kernel,
        out_type=[
            jax.ShapeDtypeStruct((_NUM_SC, npad, d), jnp.float32),
            jax.ShapeDtypeStruct((_NUM_SC, npad), jnp.float32),
        ],
        mesh=_sc_mesh(),
        scratch_types=[
            pltpu.VMEM((max(rem, 8),), jnp.int32),
            pltpu.VMEM((rc, d), jnp.float32),
            pltpu.VMEM((dc,), jnp.float32),
            pltpu.VMEM_SHARED((npad, d), jnp.float32),
            pltpu.VMEM_SHARED((npad,), jnp.float32),
        ] + [pltpu.VMEM((c, d), jnp.float32)] * _NBUF
          + [pltpu.VMEM((c,), jnp.int32)] * _NBUF
          + [pltpu.VMEM((c,), jnp.float32)] * _NBUF
          + [pltpu.SemaphoreType.DMA] * (3 * _NBUF),
    )
    def sc_scatter(mw_hbm, ex_hbm, dst_hbm, acc_hbm, den_hbm,
                   didx_r, rbuf, dbuf, acc_sh, den_sh, *bufsem):
        mws = bufsem[:_NBUF]
        didxs = bufsem[_NBUF:2 * _NBUF]
        exs = bufsem[2 * _NBUF:3 * _NBUF]
        lsem = bufsem[3 * _NBUF:4 * _NBUF]
        isem = bufsem[4 * _NBUF:5 * _NBUF]
        ssem = bufsem[5 * _NBUF:6 * _NBUF]
        cid = lax.axis_index("c")
        sid = lax.axis_index("s")
        wid = sid * _NUM_SC + cid
        base0 = wid * epw
        zv = jnp.zeros((16,), jnp.float32)

        def zrow(i, carry):
            for k in range(d // 16):
                rbuf[i, pl.ds(k * 16, 16)] = zv
            return carry

        lax.fori_loop(0, rc, zrow, 0)

        def zbody(t, carry):
            pltpu.sync_copy(rbuf, acc_sh.at[pl.ds(sid * npt + t * rc, rc)])
            return carry

        lax.fori_loop(0, nrc, zbody, 0)

        @pl.when(sid == 0)
        def _():
            def zd(i, carry):
                dbuf[pl.ds(i * 16, 16)] = zv
                return carry

            lax.fori_loop(0, dc // 16, zd, 0)

            def zden(k, carry):
                pltpu.sync_copy(dbuf, den_sh.at[pl.ds(k * dc, dc)])
                return carry

            lax.fori_loop(0, ndc, zden, 0)

        plsc.subcore_barrier()

        def group(g, carry):
            for b in range(_NBUF):
                j = g * _NBUF + b
                sl = pl.ds(base0 + j * c, c)

                @pl.when(g > 0)
                def _(b=b):
                    pltpu.make_async_copy(mws[b], acc_sh.at[didxs[b]],
                                          ssem[b]).wait()
                    pltpu.make_async_copy(exs[b], den_sh.at[didxs[b]],
                                          ssem[b]).wait()

                pltpu.async_copy(dst_hbm.at[sl], didxs[b], isem[b])
                pltpu.async_copy(mw_hbm.at[sl], mws[b], lsem[b])
                pltpu.async_copy(ex_hbm.at[sl], exs[b], lsem[b])
            for b in range(_NBUF):
                j = g * _NBUF + b
                sl = pl.ds(base0 + j * c, c)
                pltpu.make_async_copy(dst_hbm.at[sl], didxs[b], isem[b]).wait()
                pltpu.make_async_copy(mw_hbm.at[sl], mws[b], lsem[b]).wait()
                pltpu.make_async_copy(ex_hbm.at[sl], exs[b], lsem[b]).wait()
                pltpu.async_copy(mws[b], acc_sh.at[didxs[b]], ssem[b],
                                 add=True)
                pltpu.async_copy(exs[b], den_sh.at[didxs[b]], ssem[b],
                                 add=True)
            return carry

        lax.fori_loop(0, ngrp, group, 0)
        for b in range(_NBUF):
            pltpu.make_async_copy(mws[b], acc_sh.at[didxs[b]], ssem[b]).wait()
            pltpu.make_async_copy(exs[b], den_sh.at[didxs[b]], ssem[b]).wait()

        for b in range(nleft):
            j = ngrp * _NBUF + b
            sl = pl.ds(base0 + j * c, c)
            pltpu.sync_copy(dst_hbm.at[sl], didxs[b])
            pltpu.sync_copy(mw_hbm.at[sl], mws[b])
            pltpu.sync_copy(ex_hbm.at[sl], exs[b])
            pltpu.sync_copy(mws[b], acc_sh.at[didxs[b]], add=True)
            pltpu.sync_copy(exs[b], den_sh.at[didxs[b]], add=True)

        if rem:
            slr = pl.ds(base0 + nf * c, rem)
            mr = mws[0].at[pl.ds(0, rem)]
            xr = exs[0].at[pl.ds(0, rem)]
            pltpu.sync_copy(dst_hbm.at[slr], didx_r)
            pltpu.sync_copy(mw_hbm.at[slr], mr)
            pltpu.sync_copy(ex_hbm.at[slr], xr)
            pltpu.sync_copy(mr, acc_sh.at[didx_r], add=True)
            pltpu.sync_copy(xr, den_sh.at[didx_r], add=True)

        plsc.subcore_barrier()

        def ebody(t, carry):
            off = sid * npt + t * rc
            pltpu.sync_copy(acc_sh.at[pl.ds(off, rc)], rbuf)
            pltpu.sync_copy(rbuf, acc_hbm.at[cid, pl.ds(off, rc)])
            return carry

        lax.fori_loop(0, nrc, ebody, 0)

        @pl.when(sid == 0)
        def _():
            def eden(k, carry):
                sl = pl.ds(k * dc, dc)
                pltpu.sync_copy(den_sh.at[sl], dbuf)
                pltpu.sync_copy(dbuf, den_hbm.at[cid, sl])
                return carry

            lax.fori_loop(0, ndc, eden, 0)

    return sc_scatter


def kernel(x, edge_index, edge_attr, W1, W2, att_l, att_r, bias):
    n, d_in = x.shape
    e = edge_index.shape[1]
    d_e = edge_attr.shape[1]
    d_out = W1.shape[1]

    src = edge_index[0]
    dst = edge_index[1]
    w1a = W1[:d_in]
    w1b = W1[d_in:]

    nb = 1000
    p, sr3 = _tc_node(x, w1a, att_r, nb)
    sr = sr3.reshape(n)

    c = 128
    pj, srd = _make_sc_gather(n, e, d_out, c)(p, sr, src, dst)

    eb = 8000
    g = e // eb
    mw, ex3 = _tc_edge(pj, edge_attr, srd.reshape(g, 1, eb), w1b, att_l, eb)
    ex = ex3.reshape(e)

    npad = 10240
    accp, denp = _make_sc_scatter(npad, e, d_out, 64)(mw, ex, dst)

    fb = 1280
    out = _tc_final(
        accp, denp.reshape(_NUM_SC, npad // fb, 1, fb), W2,
        bias.reshape(1, d_out), fb,
    )
    return out[:n]


# speedup vs baseline: 17.2009x; 1.1643x over previous
import functools

import jax
import jax.numpy as jnp
from jax import lax
from jax.experimental import pallas as pl
from jax.experimental.pallas import tpu as pltpu
from jax.experimental.pallas import tpu_sc as plsc

_SLOPE = 0.01
_NUM_SC = 2
_NUM_TILES = 16
_NW = _NUM_SC * _NUM_TILES


def _leaky(v):
    return jnp.maximum(v, _SLOPE * v)


def _node_body(x_ref, w1a_ref, attr_ref, p_ref, sr_ref):
    xb = x_ref[...]
    p_ref[...] = jnp.dot(xb, w1a_ref[...], preferred_element_type=jnp.float32)
    sr_ref[0, 0, :] = jnp.sum(xb * attr_ref[...], axis=1)


def _edge_body(pj_ref, ea_ref, srd_ref, w1b_ref, attl_ref, mw_ref, ex_ref):
    q = jax.lax.dot_general(
        ea_ref[...], w1b_ref[...], (((0,), (0,)), ((), ())),
        preferred_element_type=jnp.float32,
    )
    m = _leaky(pj_ref[...] + q)
    aj = jax.lax.dot_general(
        attl_ref[...], m, (((1,), (1,)), ((), ())),
        preferred_element_type=jnp.float32,
    )
    alpha = _leaky(aj[0, :] + srd_ref[0, 0, :])
    ex = jnp.exp(alpha)
    ex_ref[0, 0, :] = ex
    mw_ref[...] = m * ex[:, None]


def _final_body(acc_ref, den_ref, w2_ref, b_ref, out_ref):
    acc = acc_ref[0] + acc_ref[1]
    den = den_ref[0, 0, 0, :] + den_ref[1, 0, 0, :]
    s = acc / (den + 1e-16)[:, None]
    out_ref[...] = (
        jnp.dot(s, w2_ref[...], preferred_element_type=jnp.float32) + b_ref[...]
    )


def _tc_node(x, w1a, att_r, nb):
    n, d = x.shape
    g = n // nb
    return pl.pallas_call(
        _node_body,
        grid=(g,),
        in_specs=[
            pl.BlockSpec((nb, d), lambda i: (i, 0)),
            pl.BlockSpec((d, d), lambda i: (0, 0)),
            pl.BlockSpec((1, d), lambda i: (0, 0)),
        ],
        out_specs=[
            pl.BlockSpec((nb, d), lambda i: (i, 0)),
            pl.BlockSpec((1, 1, nb), lambda i: (i, 0, 0)),
        ],
        out_shape=[
            jax.ShapeDtypeStruct((n, d), jnp.float32),
            jax.ShapeDtypeStruct((g, 1, nb), jnp.float32),
        ],
    )(x, w1a, att_r)


def _tc_edge(pj, ea_t, srd3, w1b, attl, eb):
    e, d = pj.shape
    de = ea_t.shape[0]
    g = e // eb
    return pl.pallas_call(
        _edge_body,
        grid=(g,),
        in_specs=[
            pl.BlockSpec((eb, d), lambda i: (i, 0)),
            pl.BlockSpec((de, eb), lambda i: (0, i)),
            pl.BlockSpec((1, 1, eb), lambda i: (i, 0, 0)),
            pl.BlockSpec((de, d), lambda i: (0, 0)),
            pl.BlockSpec((1, d), lambda i: (0, 0)),
        ],
        out_specs=[
            pl.BlockSpec((eb, d), lambda i: (i, 0)),
            pl.BlockSpec((1, 1, eb), lambda i: (i, 0, 0)),
        ],
        out_shape=[
            jax.ShapeDtypeStruct((e, d), jnp.float32),
            jax.ShapeDtypeStruct((g, 1, eb), jnp.float32),
        ],
    )(pj, ea_t, srd3, w1b, attl)


def _tc_final(accp, denp4, w2, bias2, nb):
    _, n, d = accp.shape
    g = n // nb
    return pl.pallas_call(
        _final_body,
        grid=(g,),
        in_specs=[
            pl.BlockSpec((2, nb, d), lambda i: (0, i, 0)),
            pl.BlockSpec((2, 1, 1, nb), lambda i: (0, i, 0, 0)),
            pl.BlockSpec((d, d), lambda i: (0, 0)),
            pl.BlockSpec((1, d), lambda i: (0, 0)),
        ],
        out_specs=pl.BlockSpec((nb, d), lambda i: (i, 0)),
        out_shape=jax.ShapeDtypeStruct((n, d), jnp.float32),
    )(accp, denp4, w2, bias2)


def _sc_mesh():
    return plsc.VectorSubcoreMesh(
        core_axis_name="c", subcore_axis_name="s",
        num_cores=_NUM_SC, num_subcores=_NUM_TILES,
    )


_NBUF = 4


@functools.lru_cache(maxsize=None)
def _make_sc_gather(n, e, d, c):
    epw = e // _NW
    nf = epw // c
    rem = epw - nf * c
    ngrp = nf // _NBUF
    nleft = nf - ngrp * _NBUF
    assert rem % 8 == 0

    @functools.partial(
        pl.kernel,
        out_type=[
            jax.ShapeDtypeStruct((e, d), jnp.float32),
            jax.ShapeDtypeStruct((e,), jnp.float32),
        ],
        mesh=_sc_mesh(),
        scratch_types=[
            pltpu.VMEM((epw,), jnp.int32),
            pltpu.VMEM((epw,), jnp.int32),
            pltpu.VMEM((epw,), jnp.float32),
        ] + [pltpu.VMEM((c, d), jnp.float32)] * _NBUF
          + [pltpu.SemaphoreType.DMA] * (2 * _NBUF + 1),
    )
    def sc_gather(p_hbm, sr_hbm, src_hbm, dst_hbm, pj_hbm, srd_hbm,
                  sidx_all, didx_all, srv_all, *bufsem):
        rows = bufsem[:_NBUF]
        gsem = bufsem[_NBUF:2 * _NBUF]
        wsem = bufsem[2 * _NBUF:3 * _NBUF]
        ssem = bufsem[3 * _NBUF]
        wid = lax.axis_index("s") * _NUM_SC + lax.axis_index("c")
        base0 = wid * epw

        pltpu.sync_copy(src_hbm.at[pl.ds(base0, epw)], sidx_all)
        pltpu.sync_copy(dst_hbm.at[pl.ds(base0, epw)], didx_all)

        def sr_fire(j, carry):
            sl = pl.ds(j * c, c)
            pltpu.async_copy(sr_hbm.at[didx_all.at[sl]], srv_all.at[sl], ssem)
            return carry

        lax.fori_loop(0, nf, sr_fire, 0)
        if rem:
            sl = pl.ds(nf * c, rem)
            pltpu.async_copy(sr_hbm.at[didx_all.at[sl]], srv_all.at[sl], ssem)

        def group(g, carry):
            for b in range(_NBUF):
                j = g * _NBUF + b

                @pl.when(g > 0)
                def _(b=b, j=j):
                    pltpu.make_async_copy(
                        rows[b],
                        pj_hbm.at[pl.ds(base0 + (j - _NBUF) * c, c)],
                        wsem[b],
                    ).wait()

                pltpu.async_copy(
                    p_hbm.at[sidx_all.at[pl.ds(j * c, c)]], rows[b], gsem[b]
                )
            for b in range(_NBUF):
                j = g * _NBUF + b
                pltpu.make_async_copy(
                    p_hbm.at[sidx_all.at[pl.ds(j * c, c)]], rows[b], gsem[b]
                ).wait()
                pltpu.async_copy(
                    rows[b], pj_hbm.at[pl.ds(base0 + j * c, c)], wsem[b]
                )
            return carry

        lax.fori_loop(0, ngrp, group, 0)
        for b in range(_NBUF):
            j = (ngrp - 1) * _NBUF + b
            pltpu.make_async_copy(
                rows[b], pj_hbm.at[pl.ds(base0 + j * c, c)], wsem[b]
            ).wait()

        for b in range(nleft):
            j = ngrp * _NBUF + b
            pltpu.async_copy(
                p_hbm.at[sidx_all.at[pl.ds(j * c, c)]], rows[b], gsem[b]
            )
        for b in range(nleft):
            j = ngrp * _NBUF + b
            pltpu.make_async_copy(
                p_hbm.at[sidx_all.at[pl.ds(j * c, c)]], rows[b], gsem[b]
            ).wait()
            pltpu.sync_copy(rows[b], pj_hbm.at[pl.ds(base0 + j * c, c)])

        if rem:
            slr = sidx_all.at[pl.ds(nf * c, rem)]
            rr = rows[0].at[pl.ds(0, rem)]
            pltpu.async_copy(p_hbm.at[slr], rr, gsem[0]).wait()
            pltpu.sync_copy(rr, pj_hbm.at[pl.ds(base0 + nf * c, rem)])

        pltpu.make_async_copy(sr_hbm.at[didx_all], srv_all, ssem).wait()
        pltpu.sync_copy(srv_all, srd_hbm.at[pl.ds(base0, epw)])

    return sc_gather


@functools.lru_cache(maxsize=None)
def _make_sc_scatter(npad, e, d, c):
    epw = e // _NW
    npt = npad // _NUM_TILES
    rc = 64
    nrc = npt // rc
    dc = 1280
    ndc = npad // dc

    nf = epw // c
    rem = epw - nf * c
    ngrp = nf // _NBUF
    nleft = nf - ngrp * _NBUF
    assert rem % 8 == 0

    @functools.partial(
        pl.kernel,
        out_type=[
            jax.ShapeDtypeStruct((_NUM_SC, npad, d), jnp.float32),
            jax.ShapeDtypeStruct((_NUM_SC, npad), jnp.float32),
        ],
        mesh=_sc_mesh(),
        scratch_types=[
            pltpu.VMEM((max(rem, 8),), jnp.int32),
            pltpu.VMEM((rc, d), jnp.float32),
            pltpu.VMEM((dc,), jnp.float32),
            pltpu.VMEM_SHARED((npad, d), jnp.float32),
            pltpu.VMEM_SHARED((npad,), jnp.float32),
        ] + [pltpu.VMEM((c, d), jnp.float32)] * _NBUF
          + [pltpu.VMEM((c,), jnp.int32)] * _NBUF
          + [pltpu.VMEM((c,), jnp.float32)] * _NBUF
          + [pltpu.SemaphoreType.DMA] * (3 * _NBUF),
    )
    def sc_scatter(mw_hbm, ex_hbm, dst_hbm, acc_hbm, den_hbm,
                   didx_r, rbuf, dbuf, acc_sh, den_sh, *bufsem):
        mws = bufsem[:_NBUF]
        didxs = bufsem[_NBUF:2 * _NBUF]
        exs = bufsem[2 * _NBUF:3 * _NBUF]
        lsem = bufsem[3 * _NBUF:4 * _NBUF]
        isem = bufsem[4 * _NBUF:5 * _NBUF]
        ssem = bufsem[5 * _NBUF:6 * _NBUF]
        cid = lax.axis_index("c")
        sid = lax.axis_index("s")
        wid = sid * _NUM_SC + cid
        base0 = wid * epw
        zv = jnp.zeros((16,), jnp.float32)

        def zrow(i, carry):
            for k in range(d // 16):
                rbuf[i, pl.ds(k * 16, 16)] = zv
            return carry

        lax.fori_loop(0, rc, zrow, 0)

        def zbody(t, carry):
            pltpu.sync_copy(rbuf, acc_sh.at[pl.ds(sid * npt + t * rc, rc)])
            return carry

        lax.fori_loop(0, nrc, zbody, 0)

        @pl.when(sid == 0)
        def _():
            def zd(i, carry):
                dbuf[pl.ds(i * 16, 16)] = zv
                return carry

            lax.fori_loop(0, dc // 16, zd, 0)

            def zden(k, carry):
                pltpu.sync_copy(dbuf, den_sh.at[pl.ds(k * dc, dc)])
                return carry

            lax.fori_loop(0, ndc, zden, 0)

        plsc.subcore_barrier()

        def group(g, carry):
            for b in range(_NBUF):
                j = g * _NBUF + b
                sl = pl.ds(base0 + j * c, c)

                @pl.when(g > 0)
                def _(b=b):
                    pltpu.make_async_copy(mws[b], acc_sh.at[didxs[b]],
                                          ssem[b]).wait()
                    pltpu.make_async_copy(exs[b], den_sh.at[didxs[b]],
                                          ssem[b]).wait()

                pltpu.async_copy(dst_hbm.at[sl], didxs[b], isem[b])
                pltpu.async_copy(mw_hbm.at[sl], mws[b], lsem[b])
                pltpu.async_copy(ex_hbm.at[sl], exs[b], lsem[b])
            for b in range(_NBUF):
                j = g * _NBUF + b
                sl = pl.ds(base0 + j * c, c)
                pltpu.make_async_copy(dst_hbm.at[sl], didxs[b], isem[b]).wait()
                pltpu.make_async_copy(mw_hbm.at[sl], mws[b], lsem[b]).wait()
                pltpu.make_async_copy(ex_hbm.at[sl], exs[b], lsem[b]).wait()
                pltpu.async_copy(mws[b], acc_sh.at[didxs[b]], ssem[b],
                                 add=True)
                pltpu.async_copy(exs[b], den_sh.at[didxs[b]], ssem[b],
                                 add=True)
            return carry

        lax.fori_loop(0, ngrp, group, 0)
        for b in range(_NBUF):
            pltpu.make_async_copy(mws[b], acc_sh.at[didxs[b]], ssem[b]).wait()
            pltpu.make_async_copy(exs[b], den_sh.at[didxs[b]], ssem[b]).wait()

        for b in range(nleft):
            j = ngrp * _NBUF + b
            sl = pl.ds(base0 + j * c, c)
            pltpu.sync_copy(dst_hbm.at[sl], didxs[b])
            pltpu.sync_copy(mw_hbm.at[sl], mws[b])
            pltpu.sync_copy(ex_hbm.at[sl], exs[b])
            pltpu.sync_copy(mws[b], acc_sh.at[didxs[b]], add=True)
            pltpu.sync_copy(exs[b], den_sh.at[didxs[b]], add=True)

        if rem:
            slr = pl.ds(base0 + nf * c, rem)
            mr = mws[0].at[pl.ds(0, rem)]
            xr = exs[0].at[pl.ds(0, rem)]
            pltpu.sync_copy(dst_hbm.at[slr], didx_r)
            pltpu.sync_copy(mw_hbm.at[slr], mr)
            pltpu.sync_copy(ex_hbm.at[slr], xr)
            pltpu.sync_copy(mr, acc_sh.at[didx_r], add=True)
            pltpu.sync_copy(xr, den_sh.at[didx_r], add=True)

        plsc.subcore_barrier()

        def ebody(t, carry):
            off = sid * npt + t * rc
            pltpu.sync_copy(acc_sh.at[pl.ds(off, rc)], rbuf)
            pltpu.sync_copy(rbuf, acc_hbm.at[cid, pl.ds(off, rc)])
            return carry

        lax.fori_loop(0, nrc, ebody, 0)

        @pl.when(sid == 0)
        def _():
            def eden(k, carry):
                sl = pl.ds(k * dc, dc)
                pltpu.sync_copy(den_sh.at[sl], dbuf)
                pltpu.sync_copy(dbuf, den_hbm.at[cid, sl])
                return carry

            lax.fori_loop(0, ndc, eden, 0)

    return sc_scatter


def kernel(x, edge_index, edge_attr, W1, W2, att_l, att_r, bias):
    n, d_in = x.shape
    e = edge_index.shape[1]
    d_e = edge_attr.shape[1]
    d_out = W1.shape[1]

    src = edge_index[0]
    dst = edge_index[1]
    w1a = W1[:d_in]
    w1b = W1[d_in:]

    nb = 1000
    p, sr3 = _tc_node(x, w1a, att_r, nb)
    sr = sr3.reshape(n)

    c = 128
    pj, srd = _make_sc_gather(n, e, d_out, c)(p, sr, src, dst)

    eb = 6400
    g = e // eb
    mw, ex3 = _tc_edge(pj, edge_attr.T, srd.reshape(g, 1, eb), w1b, att_l, eb)
    ex = ex3.reshape(e)

    npad = 10240
    accp, denp = _make_sc_scatter(npad, e, d_out, 64)(mw, ex, dst)

    fb = 1280
    out = _tc_final(
        accp, denp.reshape(_NUM_SC, npad // fb, 1, fb), W2,
        bias.reshape(1, d_out), fb,
    )
    return out[:n]
